# baseline (speedup 1.0000x reference)
import jax
import jax.numpy as jnp
from jax import lax
from jax.experimental import pallas as pl
from jax.experimental.pallas import tpu as pltpu


def _conv_body(n, c_in, c_out, kh, kw, w_lanes, w_out,
               x0_ref, x1_ref, x2_ref, w_ref, ind_ref, o_ref):
    window = jnp.concatenate([x0_ref[...], x1_ref[...], x2_ref[...]], axis=1)
    m = w_out * n
    taps = []
    for i in range(kh):
        for j in range(kw):
            s = (i * w_lanes + j) * n
            taps.append(window[:, s:s + m])
    taps.append(ind_ref[...])
    patch = jnp.concatenate(taps, axis=0)

    acc_t = lax.dot_general(
        patch, w_ref[...],
        dimension_numbers=(((0,), (1,)), ((), ())),
        preferred_element_type=jnp.float32)

    o_ref[...] = acc_t.reshape(1, w_out, n, c_out)


def kernel(x, weight, bias):
    n, c_in, h, w = x.shape
    c_out, c_in_w, kh, kw = weight.shape
    h_out = h - kh + 1
    w_out = w - kw + 1
    hw = h * w
    k = c_in * kh * kw

    x_il = (x.astype(jnp.bfloat16).reshape(n, c_in, hw)
            .transpose(1, 2, 0).reshape(c_in, hw * n))
    w_mat = jnp.transpose(weight, (0, 2, 3, 1)).reshape(c_out, k)
    w_mat = w_mat.astype(jnp.bfloat16)
    q = jnp.arange(w_out * n, dtype=jnp.int32)
    ind = (q // n == jnp.arange(w, dtype=jnp.int32)[:, None])
    ind = ind.astype(jnp.bfloat16)
    b_pad = jnp.pad(bias.reshape(c_out, w_out).astype(jnp.bfloat16),
                    ((0, 0), (0, w - w_out)))
    w_aug = jnp.concatenate([w_mat, b_pad], axis=1)

    grid = (h_out,)
    blk = w * n

    def body(x0, x1, x2, w_ref, ind_ref, o_ref):
        _conv_body(n, c_in, c_out, kh, kw, w, w_out,
                   x0, x1, x2, w_ref, ind_ref, o_ref)

    out = pl.pallas_call(
        body,
        out_shape=jax.ShapeDtypeStruct((h_out, w_out, n, c_out), jnp.float32),
        grid=grid,
        in_specs=[
            pl.BlockSpec((c_in, blk), lambda hh: (0, hh)),
            pl.BlockSpec((c_in, blk), lambda hh: (0, hh + 1)),
            pl.BlockSpec((c_in, blk), lambda hh: (0, hh + 2)),
            pl.BlockSpec((c_out, k + w), lambda hh: (0, 0)),
            pl.BlockSpec((w, w_out * n), lambda hh: (0, 0)),
        ],
        out_specs=pl.BlockSpec((1, w_out, n, c_out), lambda hh: (hh, 0, 0, 0)),
        compiler_params=pltpu.CompilerParams(
            dimension_semantics=("parallel",),
            vmem_limit_bytes=60 * 1024 * 1024),
    )(x_il, x_il, x_il, w_aug, ind)

    return out.transpose(2, 3, 0, 1)


# speedup vs baseline: 1.0153x; 1.0153x over previous
import jax
import jax.numpy as jnp
from jax import lax
from jax.experimental import pallas as pl
from jax.experimental.pallas import tpu as pltpu


def _conv_body(n, c_in, c_out, kh, kw, w_lanes, w_out,
               x0_ref, x1_ref, x2_ref, w_ref, ind_ref, o_ref):
    m = w_out * n
    taps = []
    for i, xr in enumerate((x0_ref, x1_ref, x2_ref)[:kh]):
        xv = xr[...]
        for j in range(kw):
            taps.append(xv[:, j * n:j * n + m])
    taps.append(ind_ref[...])
    patch = jnp.concatenate(taps, axis=0)

    acc_t = lax.dot_general(
        patch, w_ref[...],
        dimension_numbers=(((0,), (1,)), ((), ())),
        preferred_element_type=jnp.float32)

    o_ref[...] = acc_t.reshape(1, w_out, n, c_out)


def kernel(x, weight, bias):
    n, c_in, h, w = x.shape
    c_out, c_in_w, kh, kw = weight.shape
    h_out = h - kh + 1
    w_out = w - kw + 1
    hw = h * w
    k = c_in * kh * kw

    x_il = (x.astype(jnp.bfloat16).reshape(n, c_in, hw)
            .transpose(1, 2, 0).reshape(c_in, hw * n))
    w_mat = jnp.transpose(weight, (0, 2, 3, 1)).reshape(c_out, k)
    w_mat = w_mat.astype(jnp.bfloat16)
    q = jnp.arange(w_out * n, dtype=jnp.int32)
    ind = (q // n == jnp.arange(w, dtype=jnp.int32)[:, None])
    ind = ind.astype(jnp.bfloat16)
    b_pad = jnp.pad(bias.reshape(c_out, w_out).astype(jnp.bfloat16),
                    ((0, 0), (0, w - w_out)))
    w_aug = jnp.concatenate([w_mat, b_pad], axis=1)

    grid = (h_out,)
    blk = w * n

    def body(x0, x1, x2, w_ref, ind_ref, o_ref):
        _conv_body(n, c_in, c_out, kh, kw, w, w_out,
                   x0, x1, x2, w_ref, ind_ref, o_ref)

    out = pl.pallas_call(
        body,
        out_shape=jax.ShapeDtypeStruct((h_out, w_out, n, c_out), jnp.float32),
        grid=grid,
        in_specs=[
            pl.BlockSpec((c_in, blk), lambda hh: (0, hh)),
            pl.BlockSpec((c_in, blk), lambda hh: (0, hh + 1)),
            pl.BlockSpec((c_in, blk), lambda hh: (0, hh + 2)),
            pl.BlockSpec((c_out, k + w), lambda hh: (0, 0)),
            pl.BlockSpec((w, w_out * n), lambda hh: (0, 0)),
        ],
        out_specs=pl.BlockSpec((1, w_out, n, c_out), lambda hh: (hh, 0, 0, 0)),
        compiler_params=pltpu.CompilerParams(
            dimension_semantics=("parallel",),
            vmem_limit_bytes=60 * 1024 * 1024),
    )(x_il, x_il, x_il, w_aug, ind)

    return out.transpose(2, 3, 0, 1)
